# Initial kernel scaffold; baseline (speedup 1.0000x reference)
import jax
import jax.numpy as jnp
from jax.experimental import pallas as pl


def kernel(x, edge_index, edge_attr, c1_fw, c1_fb, c1_sw, c1_sb, c1_uw, c1_ub, c2_fw, c2_fb, c2_sw, c2_sb, c2_uw, c2_ub, c3_w, c3_b):
    raise NotImplementedError("write your pallas kernel here")


# speedup vs baseline: 2.2083x; 2.2083x over previous
import functools

import jax
import jax.numpy as jnp
from jax import lax
from jax.experimental import pallas as pl
from jax.experimental.pallas import tpu as pltpu
from jax.experimental.pallas import tpu_sc as plsc

NC = 2
NS = 16
NW = NC * NS
CHUNK = 80

_HI = lax.Precision.HIGHEST


def _sc_dual_gather(tab_a, idx_a, tab_b, idx_b):
    E = idx_a.shape[0]
    D = tab_a.shape[1]
    per_w = E // NW
    assert E % NW == 0 and per_w % CHUNK == 0
    nchunk = per_w // CHUNK
    mesh = plsc.VectorSubcoreMesh(core_axis_name="c", subcore_axis_name="s")

    @functools.partial(
        pl.kernel,
        out_type=(jax.ShapeDtypeStruct((E, D), jnp.float32),
                  jax.ShapeDtypeStruct((E, D), jnp.float32)),
        mesh=mesh,
        compiler_params=pltpu.CompilerParams(use_tc_tiling_on_sc=False),
        scratch_types=[
            pltpu.VMEM((CHUNK,), jnp.int32),
            pltpu.VMEM((CHUNK,), jnp.int32),
            pltpu.VMEM((CHUNK, D), jnp.float32),
            pltpu.VMEM((CHUNK, D), jnp.float32),
            pltpu.SemaphoreType.DMA,
            pltpu.SemaphoreType.DMA,
        ],
    )
    def k(ta, ia, tb, ib, oa, ob, iva, ivb, bva, bvb, sema, semb):
        wid = lax.axis_index("s") * NC + lax.axis_index("c")
        base = wid * per_w

        @pl.loop(0, nchunk)
        def _(ci):
            off = base + ci * CHUNK
            pltpu.sync_copy(ia.at[pl.ds(off, CHUNK)], iva)
            pltpu.sync_copy(ib.at[pl.ds(off, CHUNK)], ivb)
            cpa = pltpu.async_copy(ta.at[iva], bva, sema)
            cpb = pltpu.async_copy(tb.at[ivb], bvb, semb)
            cpa.wait()
            cpb.wait()
            pltpu.sync_copy(bva, oa.at[pl.ds(off, CHUNK)])
            pltpu.sync_copy(bvb, ob.at[pl.ds(off, CHUNK)])

    return k(tab_a, idx_a, tab_b, idx_b)


def _sc_segment_sum(m, dst, zeros_nd):
    E, D = m.shape
    n_rows = zeros_nd.shape[0]
    per_w = E // NW
    assert E % NW == 0 and per_w % CHUNK == 0
    assert n_rows % NS == 0
    rps = n_rows // NS
    nchunk = per_w // CHUNK
    mesh = plsc.VectorSubcoreMesh(core_axis_name="c", subcore_axis_name="s")

    @functools.partial(
        pl.kernel,
        out_type=jax.ShapeDtypeStruct((NC, n_rows, D), jnp.float32),
        mesh=mesh,
        compiler_params=pltpu.CompilerParams(use_tc_tiling_on_sc=False),
        scratch_types=[
            pltpu.VMEM((CHUNK,), jnp.int32),
            pltpu.VMEM((CHUNK, D), jnp.float32),
            pltpu.VMEM_SHARED((n_rows, D), jnp.float32),
            pltpu.SemaphoreType.DMA,
        ],
    )
    def k(m_hbm, dst_hbm, z_hbm, out_hbm, idx_v, m_v, acc_sh, sem):
        cid = lax.axis_index("c")
        sid = lax.axis_index("s")
        wid = sid * NC + cid
        pltpu.sync_copy(z_hbm.at[pl.ds(sid * rps, rps)],
                        acc_sh.at[pl.ds(sid * rps, rps)])
        plsc.subcore_barrier()

        @pl.loop(0, nchunk)
        def _(ci):
            off = wid * per_w + ci * CHUNK
            pltpu.sync_copy(dst_hbm.at[pl.ds(off, CHUNK)], idx_v)
            pltpu.sync_copy(m_hbm.at[pl.ds(off, CHUNK)], m_v)
            pltpu.sync_copy(m_v, acc_sh.at[idx_v], add=True)

        plsc.subcore_barrier()
        pltpu.sync_copy(acc_sh.at[pl.ds(sid * rps, rps)],
                        out_hbm.at[cid, pl.ds(sid * rps, rps)])

    return k(m, dst, zeros_nd)


def _tc_proj(x, wdT, wsT, uwT, ub8):
    n = x.shape[0]

    def body(x_ref, wd_ref, ws_ref, uw_ref, ub_ref, td_ref, ts_ref, u_ref):
        xv = x_ref[...]
        td_ref[...] = jnp.dot(xv, wd_ref[...], precision=_HI)
        ts_ref[...] = jnp.dot(xv, ws_ref[...], precision=_HI)
        u_ref[...] = jnp.dot(xv, uw_ref[...], precision=_HI) + ub_ref[0:1, :]

    return pl.pallas_call(
        body,
        out_shape=(jax.ShapeDtypeStruct((n, wdT.shape[1]), jnp.float32),
                   jax.ShapeDtypeStruct((n, wsT.shape[1]), jnp.float32),
                   jax.ShapeDtypeStruct((n, uwT.shape[1]), jnp.float32)),
    )(x, wdT, wsT, uwT, ub8)


def _tc_edge1(gd, gs, ea, weT, b8, blk):
    E, D2 = gd.shape
    D = D2 // 2
    ne = ea.shape[1]

    def body(gd_ref, gs_ref, ea_ref, we_ref, b_ref, o_ref):
        h = (gd_ref[...] + gs_ref[...]
             + jnp.dot(ea_ref[...], we_ref[...], precision=_HI)
             + b_ref[0:1, :])
        o_ref[...] = jax.nn.sigmoid(h[:, :D]) * jax.nn.softplus(h[:, D:])

    return pl.pallas_call(
        body,
        grid=(E // blk,),
        in_specs=[
            pl.BlockSpec((blk, D2), lambda i: (i, 0)),
            pl.BlockSpec((blk, D2), lambda i: (i, 0)),
            pl.BlockSpec((blk, ne), lambda i: (i, 0)),
            pl.BlockSpec(weT.shape, lambda i: (0, 0)),
            pl.BlockSpec(b8.shape, lambda i: (0, 0)),
        ],
        out_specs=pl.BlockSpec((blk, D), lambda i: (i, 0)),
        out_shape=jax.ShapeDtypeStruct((E, D), jnp.float32),
    )(gd, gs, ea, weT, b8)


def _tc_edge2(gd, gs, ea, wzT, b8, blk):
    E, D = gd.shape
    ne = ea.shape[1]
    Do2 = wzT.shape[1]
    Do = Do2 // 2

    def body(gd_ref, gs_ref, ea_ref, wz_ref, b_ref, o_ref):
        z = jnp.concatenate([gd_ref[...], gs_ref[...], ea_ref[...]], axis=1)
        h = jnp.dot(z, wz_ref[...], precision=_HI) + b_ref[0:1, :]
        o_ref[...] = jax.nn.sigmoid(h[:, :Do]) * jax.nn.softplus(h[:, Do:])

    return pl.pallas_call(
        body,
        grid=(E // blk,),
        in_specs=[
            pl.BlockSpec((blk, D), lambda i: (i, 0)),
            pl.BlockSpec((blk, D), lambda i: (i, 0)),
            pl.BlockSpec((blk, ne), lambda i: (i, 0)),
            pl.BlockSpec(wzT.shape, lambda i: (0, 0)),
            pl.BlockSpec(b8.shape, lambda i: (0, 0)),
        ],
        out_specs=pl.BlockSpec((blk, Do), lambda i: (i, 0)),
        out_shape=jax.ShapeDtypeStruct((E, Do), jnp.float32),
    )(gd, gs, ea, wzT, b8)


def _tc_combine(p, u):
    n, d = u.shape

    def body(p_ref, u_ref, o_ref):
        o_ref[...] = p_ref[0] + p_ref[1] + u_ref[...]

    return pl.pallas_call(
        body,
        out_shape=jax.ShapeDtypeStruct((n, d), jnp.float32),
    )(p, u)


def _tc_final(p, out1, uwT, ub8, w0T, w1T, w2T, b38, g):
    n, d16 = out1.shape
    d = uwT.shape[1]
    P = n // g

    def body(p_ref, o1_ref, uw_ref, ub_ref, w0_ref, w1_ref, w2_ref, b_ref,
             o_ref):
        t = (p_ref[0] + p_ref[1]
             + jnp.dot(o1_ref[...], uw_ref[...], precision=_HI)
             + ub_ref[0:1, :])
        w0 = w0_ref[...]
        w1 = w1_ref[...]
        w2 = w2_ref[...]
        b = b_ref[0:1, :]
        for gi in range(g):
            a = max(gi - 2, 0)
            c = max(gi - 1, 0)
            ta = t[a * P:(a + 1) * P]
            tb = t[c * P:(c + 1) * P]
            tg = t[gi * P:(gi + 1) * P]
            y = (jnp.dot(ta, w0, precision=_HI)
                 + jnp.dot(tb, w1, precision=_HI)
                 + jnp.dot(tg, w2, precision=_HI) + b)
            o_ref[gi * P:(gi + 1) * P, :] = jax.nn.relu(y) + tg

    return pl.pallas_call(
        body,
        out_shape=jax.ShapeDtypeStruct((n, d), jnp.float32),
    )(p, out1, uwT, ub8, w0T, w1T, w2T, b38)


def kernel(x, edge_index, edge_attr, c1_fw, c1_fb, c1_sw, c1_sb, c1_uw, c1_ub,
           c2_fw, c2_fb, c2_sw, c2_sb, c2_uw, c2_ub, c3_w, c3_b):
    n, ch = x.shape
    E, dim = edge_attr.shape
    g = 10
    src = edge_index[0]
    dst = edge_index[1]

    def b8(v):
        return jnp.broadcast_to(v[None, :], (8, v.shape[0]))

    wd1T = jnp.concatenate([c1_fw[:, :ch], c1_sw[:, :ch]], axis=0).T
    ws1T = jnp.concatenate([c1_fw[:, ch:2 * ch], c1_sw[:, ch:2 * ch]],
                           axis=0).T
    we1T = jnp.concatenate([c1_fw[:, 2 * ch:], c1_sw[:, 2 * ch:]], axis=0).T
    b1 = jnp.concatenate([c1_fb, c1_sb])

    td1, ts1, u1 = _tc_proj(x, wd1T, ws1T, c1_uw.T, b8(c1_ub))
    gd1, gs1 = _sc_dual_gather(td1, dst, ts1, src)
    m1 = _tc_edge1(gd1, gs1, edge_attr, we1T, b8(b1), 8000)
    zeros16 = jnp.zeros((n, m1.shape[1]), jnp.float32)
    p1 = _sc_segment_sum(m1, dst, zeros16)
    out1 = _tc_combine(p1, u1)

    wz2T = jnp.concatenate([c2_fw, c2_sw], axis=0).T
    b2 = jnp.concatenate([c2_fb, c2_sb])
    gd2, gs2 = _sc_dual_gather(out1, dst, out1, src)
    m2 = _tc_edge2(gd2, gs2, edge_attr, wz2T, b8(b2), 8000)
    zeros32 = jnp.zeros((n, m2.shape[1]), jnp.float32)
    p2 = _sc_segment_sum(m2, dst, zeros32)

    w0T = c3_w[:, :, 0, 0].T
    w1T = c3_w[:, :, 0, 1].T
    w2T = c3_w[:, :, 0, 2].T
    out = _tc_final(p2, out1, c2_uw.T, b8(c2_ub), w0T, w1T, w2T, b8(c3_b), g)
    return out


# speedup vs baseline: 3.5724x; 1.6178x over previous
import functools

import jax
import jax.numpy as jnp
from jax import lax
from jax.experimental import pallas as pl
from jax.experimental.pallas import tpu as pltpu
from jax.experimental.pallas import tpu_sc as plsc

NC = 2
NS = 16
NW = NC * NS
CHUNK = 80

_HI = lax.Precision.HIGHEST
_SC_PARAMS = pltpu.CompilerParams(use_tc_tiling_on_sc=False)


def _sc_gather4(tab_df, tab_ds, tab_sf, tab_ss, idx_d, idx_s, n, d):
    E = idx_d.shape[0]
    per_w = E // NW
    assert E % NW == 0 and per_w % CHUNK == 0
    nchunk = per_w // CHUNK
    mesh = plsc.VectorSubcoreMesh(core_axis_name="c", subcore_axis_name="s")
    out2d = jax.ShapeDtypeStruct((E, d), jnp.float32)

    @functools.partial(
        pl.kernel,
        out_type=(out2d, out2d, out2d, out2d),
        mesh=mesh,
        compiler_params=_SC_PARAMS,
        scratch_types=[
            pltpu.VMEM((CHUNK,), jnp.int32),
            pltpu.VMEM((CHUNK,), jnp.int32),
            pltpu.VMEM((CHUNK, d), jnp.float32),
            pltpu.VMEM((CHUNK, d), jnp.float32),
            pltpu.VMEM((CHUNK, d), jnp.float32),
            pltpu.VMEM((CHUNK, d), jnp.float32),
            pltpu.SemaphoreType.DMA,
            pltpu.SemaphoreType.DMA,
            pltpu.SemaphoreType.DMA,
            pltpu.SemaphoreType.DMA,
        ],
    )
    def k(tdf2, tds2, tsf2, tss2, i_d, i_s, odf2, ods2, osf2, oss2,
          ivd, ivs, b0, b1, b2, b3, s0, s1, s2, s3):
        wid = lax.axis_index("s") * NC + lax.axis_index("c")
        base = wid * per_w

        @pl.loop(0, nchunk)
        def _(ci):
            off = base + ci * CHUNK
            pltpu.sync_copy(i_d.at[pl.ds(off, CHUNK)], ivd)
            pltpu.sync_copy(i_s.at[pl.ds(off, CHUNK)], ivs)
            c0 = pltpu.async_copy(tdf2.at[ivd], b0, s0)
            c1 = pltpu.async_copy(tds2.at[ivd], b1, s1)
            c2 = pltpu.async_copy(tsf2.at[ivs], b2, s2)
            c3 = pltpu.async_copy(tss2.at[ivs], b3, s3)
            c0.wait()
            c1.wait()
            c2.wait()
            c3.wait()
            pltpu.sync_copy(b0, odf2.at[pl.ds(off, CHUNK)])
            pltpu.sync_copy(b1, ods2.at[pl.ds(off, CHUNK)])
            pltpu.sync_copy(b2, osf2.at[pl.ds(off, CHUNK)])
            pltpu.sync_copy(b3, oss2.at[pl.ds(off, CHUNK)])

    return k(tab_df, tab_ds, tab_sf, tab_ss, idx_d, idx_s)


def _sc_gather2(tab, idx_d, idx_s, n, d):
    E = idx_d.shape[0]
    per_w = E // NW
    assert E % NW == 0 and per_w % CHUNK == 0
    nchunk = per_w // CHUNK
    mesh = plsc.VectorSubcoreMesh(core_axis_name="c", subcore_axis_name="s")
    out2d = jax.ShapeDtypeStruct((E, d), jnp.float32)

    @functools.partial(
        pl.kernel,
        out_type=(out2d, out2d),
        mesh=mesh,
        compiler_params=_SC_PARAMS,
        scratch_types=[
            pltpu.VMEM((CHUNK,), jnp.int32),
            pltpu.VMEM((CHUNK,), jnp.int32),
            pltpu.VMEM((CHUNK, d), jnp.float32),
            pltpu.VMEM((CHUNK, d), jnp.float32),
            pltpu.SemaphoreType.DMA,
            pltpu.SemaphoreType.DMA,
        ],
    )
    def k(tab2, i_d, i_s, od2, os2, ivd, ivs, bd, bs, sd, ss):
        wid = lax.axis_index("s") * NC + lax.axis_index("c")
        base = wid * per_w

        @pl.loop(0, nchunk)
        def _(ci):
            off = base + ci * CHUNK
            pltpu.sync_copy(i_d.at[pl.ds(off, CHUNK)], ivd)
            pltpu.sync_copy(i_s.at[pl.ds(off, CHUNK)], ivs)
            cd = pltpu.async_copy(tab2.at[ivd], bd, sd)
            cs = pltpu.async_copy(tab2.at[ivs], bs, ss)
            cd.wait()
            cs.wait()
            pltpu.sync_copy(bd, od2.at[pl.ds(off, CHUNK)])
            pltpu.sync_copy(bs, os2.at[pl.ds(off, CHUNK)])

    return k(tab, idx_d, idx_s)


def _sc_segment_sum(m_lins, dst, zeros_lin, n, d):
    E = dst.shape[0]
    nm = len(m_lins)
    per_w = E // NW
    assert E % NW == 0 and per_w % CHUNK == 0
    assert n % NS == 0
    rps = n // NS
    nchunk = per_w // CHUNK
    mesh = plsc.VectorSubcoreMesh(core_axis_name="c", subcore_axis_name="s")
    out3d = jax.ShapeDtypeStruct((NC, n, d), jnp.float32)

    @functools.partial(
        pl.kernel,
        out_type=tuple(out3d for _ in range(nm)),
        mesh=mesh,
        compiler_params=_SC_PARAMS,
        scratch_types=(
            [pltpu.VMEM((CHUNK,), jnp.int32)]
            + [pltpu.VMEM((CHUNK, d), jnp.float32) for _ in range(nm)]
            + [pltpu.VMEM_SHARED((n, d), jnp.float32) for _ in range(nm)]
            + [pltpu.SemaphoreType.DMA]
        ),
    )
    def k(*refs):
        m_hbms = refs[:nm]
        dst_hbm = refs[nm]
        z_hbm = refs[nm + 1]
        out_hbms = refs[nm + 2:2 * nm + 2]
        idx_v = refs[2 * nm + 2]
        m_vs = refs[2 * nm + 3:3 * nm + 3]
        acc_shs = refs[3 * nm + 3:4 * nm + 3]
        m2s = m_hbms
        z2 = z_hbm
        out2s = out_hbms
        cid = lax.axis_index("c")
        sid = lax.axis_index("s")
        wid = sid * NC + cid
        for acc_sh in acc_shs:
            pltpu.sync_copy(z2.at[pl.ds(sid * rps, rps)],
                            acc_sh.at[pl.ds(sid * rps, rps)])
        plsc.subcore_barrier()

        @pl.loop(0, nchunk)
        def _(ci):
            off = wid * per_w + ci * CHUNK
            pltpu.sync_copy(dst_hbm.at[pl.ds(off, CHUNK)], idx_v)
            for m2, m_v, acc_sh in zip(m2s, m_vs, acc_shs):
                pltpu.sync_copy(m2.at[pl.ds(off, CHUNK)], m_v)
                pltpu.sync_copy(m_v, acc_sh.at[idx_v], add=True)

        plsc.subcore_barrier()
        for acc_sh, out2 in zip(acc_shs, out2s):
            pltpu.sync_copy(acc_sh.at[pl.ds(sid * rps, rps)],
                            out2.at[cid, pl.ds(sid * rps, rps)])

    return k(*m_lins, dst, zeros_lin)


def _tc_proj(x8, w8s):
    n8 = x8.shape[0]

    def body(x_ref, *refs):
        w_refs = refs[:len(w8s)]
        o_refs = refs[len(w8s):]
        xv = x_ref[...]
        for w_ref, o_ref in zip(w_refs, o_refs):
            o_ref[...] = jnp.dot(xv, w_ref[...], precision=_HI)

    return pl.pallas_call(
        body,
        out_shape=tuple(jax.ShapeDtypeStruct((n8, 128), jnp.float32)
                        for _ in w8s),
    )(x8, *w8s)


def _tc_edge1(gdf, gds, gsf, gss, ea8, cf, cs, bf, bs, blk):
    M = gdf.shape[0]

    def body(gdf_r, gds_r, gsf_r, gss_r, ea_r, cf_r, cs_r, bf_r, bs_r, o_r):
        eav = ea_r[...]
        hf = (gdf_r[...] + gsf_r[...]
              + jnp.dot(eav, cf_r[...], precision=_HI) + bf_r[0:1, :])
        hs = (gds_r[...] + gss_r[...]
              + jnp.dot(eav, cs_r[...], precision=_HI) + bs_r[0:1, :])
        o_r[...] = jax.nn.sigmoid(hf) * jax.nn.softplus(hs)

    eblk = pl.BlockSpec((blk, 128), lambda i: (i, 0))
    wblk = pl.BlockSpec((128, 128), lambda i: (0, 0))
    bblk = pl.BlockSpec((8, 128), lambda i: (0, 0))
    return pl.pallas_call(
        body,
        grid=(M // blk,),
        in_specs=[eblk, eblk, eblk, eblk, eblk, wblk, wblk, bblk, bblk],
        out_specs=eblk,
        out_shape=jax.ShapeDtypeStruct((M, 128), jnp.float32),
    )(gdf, gds, gsf, gss, ea8, cf, cs, bf, bs)


def _tc_edge2(gd8, gs8, ea8, wcat, bf_a, bf_b, bs_a, bs_b, blk):
    M = gd8.shape[0]

    def body(gd_r, gs_r, ea_r, w_r, bfa_r, bfb_r, bsa_r, bsb_r, oa_r, ob_r):
        z = jnp.concatenate([gd_r[...], gs_r[...], ea_r[...]], axis=1)
        h = jnp.dot(z, w_r[...], precision=_HI)
        hfa = h[:, 0:128] + bfa_r[0:1, :]
        hfb = h[:, 128:256] + bfb_r[0:1, :]
        hsa = h[:, 256:384] + bsa_r[0:1, :]
        hsb = h[:, 384:512] + bsb_r[0:1, :]
        oa_r[...] = jax.nn.sigmoid(hfa) * jax.nn.softplus(hsa)
        ob_r[...] = jax.nn.sigmoid(hfb) * jax.nn.softplus(hsb)

    eblk = pl.BlockSpec((blk, 128), lambda i: (i, 0))
    wblk = pl.BlockSpec((384, 512), lambda i: (0, 0))
    bblk = pl.BlockSpec((8, 128), lambda i: (0, 0))
    out = jax.ShapeDtypeStruct((M, 128), jnp.float32)
    return pl.pallas_call(
        body,
        grid=(M // blk,),
        in_specs=[eblk, eblk, eblk, wblk, bblk, bblk, bblk, bblk],
        out_specs=(eblk, eblk),
        out_shape=(out, out),
    )(gd8, gs8, ea8, wcat, bf_a, bf_b, bs_a, bs_b)


def _tc_combine(p3, u1p):
    n8 = u1p.shape[0]

    def body(p_ref, u_ref, o_ref):
        o_ref[...] = p_ref[0] + p_ref[1] + u_ref[...]

    return pl.pallas_call(
        body,
        out_shape=jax.ShapeDtypeStruct((n8, 128), jnp.float32),
    )(p3, u1p)


def _tc_final(p2, out1, uwT, ub8, w0T, w1T, w2T, b38, g):
    n = out1.shape[0]
    d = uwT.shape[1]
    P = n // g

    def body(p_ref, o1_ref, uw_ref, ub_ref, w0_ref, w1_ref, w2_ref, b_ref,
             o_ref):
        t = (p_ref[0] + p_ref[1]
             + jnp.dot(o1_ref[...], uw_ref[...], precision=_HI)
             + ub_ref[0:1, :])
        w0 = w0_ref[...]
        w1 = w1_ref[...]
        w2 = w2_ref[...]
        b = b_ref[0:1, :]
        for gi in range(g):
            a = max(gi - 2, 0)
            c = max(gi - 1, 0)
            ta = t[a * P:(a + 1) * P]
            tb = t[c * P:(c + 1) * P]
            tg = t[gi * P:(gi + 1) * P]
            y = (jnp.dot(ta, w0, precision=_HI)
                 + jnp.dot(tb, w1, precision=_HI)
                 + jnp.dot(tg, w2, precision=_HI) + b)
            o_ref[gi * P:(gi + 1) * P, :] = jax.nn.relu(y) + tg

    return pl.pallas_call(
        body,
        out_shape=jax.ShapeDtypeStruct((n, d), jnp.float32),
    )(p2, out1, uwT, ub8, w0T, w1T, w2T, b38)


def _kron8(w):
    return jnp.kron(jnp.eye(8, dtype=jnp.float32), w)


def kernel(x, edge_index, edge_attr, c1_fw, c1_fb, c1_sw, c1_sb, c1_uw, c1_ub,
           c2_fw, c2_fb, c2_sw, c2_sb, c2_uw, c2_ub, c3_w, c3_b):
    n, ch = x.shape
    E, dim = edge_attr.shape
    g = 10
    src = edge_index[0]
    dst = edge_index[1]

    def btile8(v):
        return jnp.broadcast_to(jnp.tile(v, 8)[None, :], (8, 128))

    def b8(v):
        return jnp.broadcast_to(v[None, :], (8, v.shape[0]))

    x8 = x.reshape(n // 8, 8 * ch)
    ea8 = edge_attr.reshape(E // 8, 8 * dim)

    w_dfT = c1_fw[:, :ch].T
    w_dsT = c1_sw[:, :ch].T
    w_sfT = c1_fw[:, ch:2 * ch].T
    w_ssT = c1_sw[:, ch:2 * ch].T
    w_uT = c1_uw.T
    w8s = [_kron8(w) for w in (w_dfT, w_dsT, w_sfT, w_ssT, w_uT)]
    tdf, tds, tsf, tss, u1p = _tc_proj(x8, w8s)

    gdf, gds, gsf, gss = _sc_gather4(
        tdf.reshape(n, 16), tds.reshape(n, 16), tsf.reshape(n, 16),
        tss.reshape(n, 16), dst, src, n, 16)
    cf1 = _kron8(c1_fw[:, 2 * ch:].T)
    cs1 = _kron8(c1_sw[:, 2 * ch:].T)
    m1 = _tc_edge1(gdf.reshape(E // 8, 128), gds.reshape(E // 8, 128),
                   gsf.reshape(E // 8, 128), gss.reshape(E // 8, 128),
                   ea8, cf1, cs1, btile8(c1_fb), btile8(c1_sb), 4000)

    zeros16 = jnp.zeros((n, 16), jnp.float32)
    (p1,) = _sc_segment_sum([m1.reshape(E, 16)], dst, zeros16, n, 16)
    out1p = _tc_combine(p1.reshape(NC, n // 8, 128), u1p)
    out1 = out1p.reshape(n, 16)

    gd2, gs2 = _sc_gather2(out1, dst, src, n, 16)
    wf = c2_fw.T
    ws = c2_sw.T
    blocks = []
    for wpart in (wf[:, :16], wf[:, 16:], ws[:, :16], ws[:, 16:]):
        cols = []
        for seg in range(3):
            cols.append(_kron8(wpart[seg * 16:(seg + 1) * 16, :]))
        blocks.append(jnp.concatenate(cols, axis=0))
    wcat = jnp.concatenate(blocks, axis=1)
    m2a, m2b = _tc_edge2(gd2.reshape(E // 8, 128), gs2.reshape(E // 8, 128),
                         ea8, wcat,
                         btile8(c2_fb[:16]), btile8(c2_fb[16:]),
                         btile8(c2_sb[:16]), btile8(c2_sb[16:]), 4000)

    p2a, p2b = _sc_segment_sum([m2a.reshape(E, 16), m2b.reshape(E, 16)],
                               dst, zeros16, n, 16)
    p2 = jnp.concatenate([p2a, p2b], axis=2)

    w0T = c3_w[:, :, 0, 0].T
    w1T = c3_w[:, :, 0, 1].T
    w2T = c3_w[:, :, 0, 2].T
    out = _tc_final(p2, out1, c2_uw.T, b8(c2_ub), w0T, w1T, w2T, b8(c3_b), g)
    return out
